# baseline (speedup 1.0000x reference)
import functools

import jax
import jax.numpy as jnp
from jax import lax
from jax.experimental import pallas as pl
from jax.experimental.pallas import tpu as pltpu
from jax.experimental.pallas import tpu_sc as plsc

N_NODES = 10000
N_PAD = 10240
N_EDGES = 160000
D = 256
H = D // 2
CHUNK = 128
N_TILES = 16
K_PER_TILE = 80
N_CHUNKS = N_TILES * K_PER_TILE
E_PADDED = N_CHUNKS * CHUNK
ROWS_PER_TILE = N_PAD // N_TILES
NI = 4
NR = 2


def _sc_segment_sum(xh, src_idx, dst_idx, zeros):
    mesh = plsc.VectorSubcoreMesh(core_axis_name="c", subcore_axis_name="s")

    @functools.partial(
        pl.kernel,
        out_type=jax.ShapeDtypeStruct((2 * N_PAD, H), jnp.float32),
        mesh=mesh,
        scratch_types=[
            pltpu.VMEM((NI, CHUNK), jnp.int32),
            pltpu.VMEM((NI, CHUNK), jnp.int32),
            pltpu.VMEM((NR, CHUNK, H), jnp.float32),
            pltpu.VMEM_SHARED((N_PAD, H), jnp.float32),
        ]
        + [pltpu.SemaphoreType.DMA] * (NI + 2 * NR),
    )
    def sc_kernel(xh_hbm, src_hbm, dst_hbm, zeros_hbm, out_hbm,
                  src_v, dst_v, rows_v, acc_sh, *sems):
        sem_i = sems[:NI]
        sem_g = sems[NI:NI + NR]
        sem_s = sems[NI + NR:]
        c = lax.axis_index("c")
        s = lax.axis_index("s")
        base = s * K_PER_TILE

        def i_start_q(j, q):
            pltpu.async_copy(src_hbm.at[c, base + j], src_v.at[q], sem_i[q])
            pltpu.async_copy(dst_hbm.at[base + j], dst_v.at[q], sem_i[q])

        def i_wait_q(j, q):
            pltpu.make_async_copy(
                src_hbm.at[c, base + j], src_v.at[q], sem_i[q]).wait()
            pltpu.make_async_copy(
                dst_hbm.at[base + j], dst_v.at[q], sem_i[q]).wait()

        def g_start(q, r):
            pltpu.async_copy(xh_hbm.at[src_v.at[q]], rows_v.at[r], sem_g[r])

        def g_wait(q, r):
            pltpu.make_async_copy(
                xh_hbm.at[src_v.at[q]], rows_v.at[r], sem_g[r]).wait()

        def s_start(q, r):
            pltpu.async_copy(rows_v.at[r], acc_sh.at[dst_v.at[q]],
                             sem_s[r], add=True)

        def s_wait(q, r):
            pltpu.make_async_copy(
                rows_v.at[r], acc_sh.at[dst_v.at[q]], sem_s[r]).wait()

        i_start_q(0, 0)
        i_start_q(1, 1)
        i_start_q(2, 2)
        pltpu.sync_copy(zeros_hbm, acc_sh.at[pl.ds(s * ROWS_PER_TILE, ROWS_PER_TILE)])
        i_wait_q(0, 0)
        g_start(0, 0)
        plsc.subcore_barrier()

        g_wait(0, 0); s_start(0, 0)
        i_start_q(3, 3)
        i_wait_q(1, 1); g_start(1, 1)
        g_wait(1, 1); s_start(1, 1)
        s_wait(0, 0)
        i_start_q(4, 0)
        i_wait_q(2, 2); g_start(2, 0)

        @pl.loop(0, (K_PER_TILE - 8) // NI)
        def _(g):
            for jj in range(NI):
                k = g * NI + jj + 2
                qi = (jj + 2) % NI
                ri = jj % NR
                g_wait(qi, ri)
                s_start(qi, ri)
                s_wait((jj + 1) % NI, (jj + 1) % NR)
                i_start_q(k + 3, (jj + 1) % NI)
                i_wait_q(k + 1, (jj + 3) % NI)
                g_start((jj + 3) % NI, (jj + 1) % NR)

        g_wait(2, 0); s_start(2, 0); s_wait(1, 1); i_start_q(77, 1)
        i_wait_q(75, 3); g_start(3, 1)
        g_wait(3, 1); s_start(3, 1); s_wait(2, 0); i_start_q(78, 2)
        i_wait_q(76, 0); g_start(0, 0)
        g_wait(0, 0); s_start(0, 0); s_wait(3, 1); i_start_q(79, 3)
        i_wait_q(77, 1); g_start(1, 1)
        g_wait(1, 1); s_start(1, 1); s_wait(0, 0)
        i_wait_q(78, 2); g_start(2, 0)
        g_wait(2, 0); s_start(2, 0); s_wait(1, 1)
        i_wait_q(79, 3); g_start(3, 1)
        g_wait(3, 1); s_start(3, 1); s_wait(2, 0)
        s_wait(3, 1)

        plsc.subcore_barrier()
        pltpu.sync_copy(
            acc_sh.at[pl.ds(s * ROWS_PER_TILE, ROWS_PER_TILE)],
            out_hbm.at[pl.ds(c * N_PAD + s * ROWS_PER_TILE, ROWS_PER_TILE)],
        )

    return sc_kernel(xh, src_idx, dst_idx, zeros)


def _tc_finish_body(a0_ref, a1_ref, x_ref, w0_ref, w1_ref, wr_ref, b_ref, o_ref):
    y = jnp.dot(a0_ref[...], w0_ref[...],
                preferred_element_type=jnp.float32,
                precision=jax.lax.Precision.HIGHEST)
    y = y + jnp.dot(a1_ref[...], w1_ref[...],
                    preferred_element_type=jnp.float32,
                    precision=jax.lax.Precision.HIGHEST)
    y = y + jnp.dot(x_ref[...], wr_ref[...],
                    preferred_element_type=jnp.float32,
                    precision=jax.lax.Precision.HIGHEST)
    y = y + b_ref[...]
    m = jnp.max(y, axis=-1, keepdims=True)
    t = y - m
    lse = jnp.log(jnp.sum(jnp.exp(t), axis=-1, keepdims=True))
    o_ref[...] = t - lse


def _tc_finish(agg0, agg1, x, w0, w1, wr, b2d):
    n = x.shape[0]
    blk = 1000
    return pl.pallas_call(
        _tc_finish_body,
        grid=(n // blk,),
        in_specs=[
            pl.BlockSpec((blk, H), lambda i: (i, 0)),
            pl.BlockSpec((blk, H), lambda i: (i, 0)),
            pl.BlockSpec((blk, D), lambda i: (i, 0)),
            pl.BlockSpec((H, D), lambda i: (0, 0)),
            pl.BlockSpec((H, D), lambda i: (0, 0)),
            pl.BlockSpec((D, D), lambda i: (0, 0)),
            pl.BlockSpec((1, D), lambda i: (0, 0)),
        ],
        out_specs=pl.BlockSpec((blk, D), lambda i: (i, 0)),
        out_shape=jax.ShapeDtypeStruct((n, D), jnp.float32),
    )(agg0, agg1, x, w0, w1, wr, b2d)


def kernel(x, edge_index, W_rel, W_root, b):
    src = edge_index[0]
    dst = edge_index[1]
    n_extra = E_PADDED - N_EDGES
    src_pad = jnp.concatenate([src, jnp.zeros((n_extra,), jnp.int32)])
    dst_pad = jnp.concatenate(
        [dst, N_NODES + (jnp.arange(n_extra, dtype=jnp.int32) % (N_PAD - N_NODES))])
    xh = jnp.concatenate([x[:, :H], x[:, H:]], axis=0)
    srcs = src_pad.reshape(N_CHUNKS, CHUNK)
    src_idx = jnp.stack([srcs, srcs + N_NODES])
    dst_idx = dst_pad.reshape(N_CHUNKS, CHUNK)
    zeros = jnp.zeros((ROWS_PER_TILE, H), jnp.float32)

    agg_cat = _sc_segment_sum(xh, src_idx, dst_idx, zeros)

    out = _tc_finish(
        agg_cat[:N_NODES], agg_cat[N_PAD:N_PAD + N_NODES], x,
        W_rel[:, :H].T, W_rel[:, H:].T, W_root.T, b.reshape(1, D),
    )
    return out
